# baseline (speedup 1.0000x reference)
import functools

import jax
import jax.numpy as jnp
from jax import lax
from jax.experimental import pallas as pl
from jax.experimental.pallas import tpu as pltpu
from jax.experimental.pallas import tpu_sc as plsc

B = 16384
EMB = 64
NC = 2
NS = 16
NW = NC * NS
CHUNK = 128
CHUNKS = B // NW // CHUNK
LANES = 16


def _sbr_body(uid_hbm, iid_hbm, w_hbm, ue_hbm, ub_hbm, ie_hbm, ib_hbm,
              se_hbm, mean_hbm, out_hbm,
              uidx_v, iidx_v, bu_v, bi_v, mean_v,
              U_v, I_v, S_v, W_v, prod_v, out_v, sem):
    wid = lax.axis_index("s") * NC + lax.axis_index("c")

    pltpu.sync_copy(uid_hbm.at[wid], uidx_v)
    pltpu.sync_copy(iid_hbm.at[wid], iidx_v)
    pltpu.sync_copy(mean_hbm, mean_v)

    bias_cps = []
    for c in range(CHUNKS):
        bias_cps.append(pltpu.make_async_copy(
            ub_hbm.at[uidx_v.at[c]], bu_v.at[c], sem))
        bias_cps.append(pltpu.make_async_copy(
            ib_hbm.at[iidx_v.at[c]], bi_v.at[c], sem))
    for cp in bias_cps:
        cp.start()
    for cp in bias_cps:
        cp.wait()

    lane_iota = lax.iota(jnp.int32, LANES)
    mean_vec = mean_v[...]

    for c in range(CHUNKS):
        row0 = wid * (CHUNKS * CHUNK) + c * CHUNK
        cps = [
            pltpu.make_async_copy(ue_hbm.at[uidx_v.at[c]], U_v, sem),
            pltpu.make_async_copy(ie_hbm.at[iidx_v.at[c]], I_v, sem),
            pltpu.make_async_copy(se_hbm.at[iidx_v.at[c]], S_v, sem),
            pltpu.make_async_copy(w_hbm.at[pl.ds(row0, CHUNK)], W_v, sem),
        ]
        for cp in cps:
            cp.start()
        for cp in cps:
            cp.wait()

        def group_body(g, _, c=c):
            for j in range(LANES):
                e = g * LANES + j
                p = U_v[e, pl.ds(0, LANES)] * I_v[e, pl.ds(0, LANES)]
                for k in range(1, EMB // LANES):
                    p += U_v[e, pl.ds(k * LANES, LANES)] * \
                         I_v[e, pl.ds(k * LANES, LANES)]
                for k in range(EMB // LANES):
                    p += S_v[e, pl.ds(k * LANES, LANES)] * \
                         W_v[e, pl.ds(k * LANES, LANES)]
                prod_v[j, :] = p
            acc = plsc.load_gather(
                prod_v, [lane_iota, jnp.full((LANES,), 0, jnp.int32)])
            for r in range(1, LANES):
                acc += plsc.load_gather(
                    prod_v, [lane_iota, jnp.full((LANES,), r, jnp.int32)])
            acc += bu_v[c, pl.ds(g * LANES, LANES)]
            acc += bi_v[c, pl.ds(g * LANES, LANES)]
            acc += mean_vec
            out_v[c, pl.ds(g * LANES, LANES)] = acc
            return 0

        lax.fori_loop(0, CHUNK // LANES, group_body, 0)

    pltpu.sync_copy(out_v, out_hbm.at[wid])


def kernel(u_id, i_id, UserShadow, user_emb, user_bias, item_emb, item_bias,
           shadow_i_emb, mean):
    uid3 = u_id.reshape(NW, CHUNKS, CHUNK)
    iid3 = i_id.reshape(NW, CHUNKS, CHUNK)
    ub_flat = user_bias.reshape(-1)
    ib_flat = item_bias.reshape(-1)
    mean16 = jnp.broadcast_to(mean, (LANES,))

    mesh = plsc.VectorSubcoreMesh(core_axis_name="c", subcore_axis_name="s")
    run = pl.kernel(
        _sbr_body,
        out_type=jax.ShapeDtypeStruct((NW, CHUNKS, CHUNK), jnp.float32),
        mesh=mesh,
        compiler_params=pltpu.CompilerParams(
            needs_layout_passes=False, use_tc_tiling_on_sc=False),
        scratch_types=[
            pltpu.VMEM((CHUNKS, CHUNK), jnp.int32),
            pltpu.VMEM((CHUNKS, CHUNK), jnp.int32),
            pltpu.VMEM((CHUNKS, CHUNK), jnp.float32),
            pltpu.VMEM((CHUNKS, CHUNK), jnp.float32),
            pltpu.VMEM((LANES,), jnp.float32),
            pltpu.VMEM((CHUNK, EMB), jnp.float32),
            pltpu.VMEM((CHUNK, EMB), jnp.float32),
            pltpu.VMEM((CHUNK, EMB), jnp.float32),
            pltpu.VMEM((CHUNK, EMB), jnp.float32),
            pltpu.VMEM((LANES, LANES), jnp.float32),
            pltpu.VMEM((CHUNKS, CHUNK), jnp.float32),
            pltpu.SemaphoreType.DMA,
        ],
    )
    out3 = run(uid3, iid3, UserShadow, user_emb, ub_flat, item_emb, ib_flat,
               shadow_i_emb, mean16)
    return out3.reshape(B)
